# baseline (speedup 1.0000x reference)
import functools

import jax
import jax.numpy as jnp
from jax import lax
from jax.experimental import pallas as pl
from jax.experimental.pallas import tpu as pltpu
from jax.experimental.pallas import tpu_sc as plsc

NC = 2
NS = 16
NW = NC * NS
LANES = 16

BATCH = 16384
CTX = 20
EMB = 32
VOCAB = 1_000_000
VOCAB_PAD = 1_000_064
N_BLOCKS = VOCAB_PAD // 128
N_FULL_BLOCKS = VOCAB // 128
BLK_PER_W = (N_BLOCKS + NW - 1) // NW
N_LINES = VOCAB_PAD * EMB // 128

B_PER_W = BATCH // NW
ROWS_PER_CHUNK = 4
IDX_PER_CHUNK = ROWS_PER_CHUNK * CTX
N_CHUNKS = B_PER_W // ROWS_PER_CHUNK
NBUF = 4
N_STEPS = N_CHUNKS // NBUF


def _transpose_block(in_v, out_v, zero, gs, ss):
    for d in range(LANES):
        for v0 in range(0, 128, LANES):
            g_lo = gs[d] + v0
            s_lo = ss[d] + (v0 << 5)
            g_hi = g_lo + (LANES << 7)
            s_hi = s_lo + LANES
            plsc.store_scatter(out_v, [zero, s_lo],
                               plsc.load_gather(in_v, [zero, g_lo]))
            plsc.store_scatter(out_v, [zero, s_hi],
                               plsc.load_gather(in_v, [zero, g_hi]))


def _fmt_body(tab_t_hbm, tail_hbm, lines_hbm, in_v, out_v,
              isem0, isem1, osem0, osem1):
    wid = lax.axis_index("s") * NC + lax.axis_index("c")
    iota16 = lax.iota(jnp.int32, LANES)
    isems = (isem0, isem1)
    osems = (osem0, osem1)

    def blk(k):
        return jnp.minimum(k * NW + wid, N_FULL_BLOCKS - 1)

    def issue_in(k, b):
        c = blk(k)
        pltpu.async_copy(tab_t_hbm.at[:, pl.ds(c * 128, 128)],
                         in_v.at[b], isems[b])

    for b in range(2):
        issue_in(b, b)

    zero = iota16 & 0
    e128 = iota16 << 7
    gs0, ss0 = [], []
    for d in range(LANES):
        pd = (iota16 + d) & (LANES - 1)
        gs0.append(e128 + pd)
        ss0.append((pd << 5) + iota16)

    def step(g, carry):
        zero, gs, ss = carry

        @pl.when(g >= 1)
        def _():
            for b in range(2):
                pltpu.make_async_copy(
                    out_v.at[b], lines_hbm.at[pl.ds(0, EMB)], osems[b]).wait()

        for b in range(2):
            k = g * 2 + b
            c = blk(k)
            pltpu.make_async_copy(
                tab_t_hbm.at[:, pl.ds(c * 128, 128)],
                in_v.at[b], isems[b]).wait()
            _transpose_block(in_v.at[b].reshape(1, EMB * 128),
                             out_v.at[b].reshape(1, EMB * 128),
                             zero, gs, ss)
            pltpu.async_copy(
                out_v.at[b], lines_hbm.at[pl.ds(c * EMB, EMB)], osems[b])
            issue_in(k + 2, b)
        return carry

    lax.fori_loop(0, (BLK_PER_W + 1) // 2, step,
                  (zero, tuple(gs0), tuple(ss0)))

    for b in range(2):
        pltpu.make_async_copy(
            out_v.at[b], lines_hbm.at[pl.ds(0, EMB)], osems[b]).wait()
        pltpu.make_async_copy(
            tab_t_hbm.at[:, pl.ds(0, 128)], in_v.at[b], isems[b]).wait()

    @pl.when(wid == 4)
    def _():
        pltpu.sync_copy(tail_hbm, in_v.at[0, pl.ds(0, 16)])
        pltpu.sync_copy(in_v.at[0, pl.ds(0, 16)],
                        lines_hbm.at[pl.ds(N_FULL_BLOCKS * EMB, 16)])


def _cbow_body(table_hbm, x_hbm, out_hbm, idx_v, rows_v, out_v,
               sem0, sem1, sem2, sem3):
    wid = lax.axis_index("s") * NC + lax.axis_index("c")
    sems = (sem0, sem1, sem2, sem3)

    pltpu.sync_copy(x_hbm.at[wid], idx_v)

    inv_ctx = jnp.float32(1.0 / CTX)

    for b in range(NBUF):
        pltpu.async_copy(table_hbm.at[idx_v.at[b]], rows_v.at[b], sems[b])

    def step(g, carry):
        for b in range(NBUF):
            j = g * NBUF + b
            pltpu.make_async_copy(
                table_hbm.at[idx_v.at[j]], rows_v.at[b], sems[b]).wait()
            for r in range(ROWS_PER_CHUNK):
                base = r * CTX
                lo = rows_v[b, base, pl.ds(0, LANES)]
                hi = rows_v[b, base, pl.ds(LANES, LANES)]
                for c in range(1, CTX):
                    lo = lo + rows_v[b, base + c, pl.ds(0, LANES)]
                    hi = hi + rows_v[b, base + c, pl.ds(LANES, LANES)]
                row = j * ROWS_PER_CHUNK + r
                out_v[row, pl.ds(0, LANES)] = lo * inv_ctx
                out_v[row, pl.ds(LANES, LANES)] = hi * inv_ctx

            @pl.when(g < N_STEPS - 1)
            def _():
                pltpu.async_copy(
                    table_hbm.at[idx_v.at[j + NBUF]], rows_v.at[b], sems[b])
        return carry

    lax.fori_loop(0, N_STEPS, step, 0)

    pltpu.sync_copy(out_v, out_hbm.at[wid])


def _mesh():
    return plsc.VectorSubcoreMesh(
        core_axis_name="c", subcore_axis_name="s",
        num_cores=NC, num_subcores=NS)


@jax.jit
def _cbow(x3, emb_table_t, tail_lines):
    fmt = functools.partial(
        pl.kernel,
        out_type=jax.ShapeDtypeStruct((N_LINES, 128), jnp.float32),
        mesh=_mesh(),
        scratch_types=[
            pltpu.VMEM((2, EMB, 128), jnp.float32),
            pltpu.VMEM((2, EMB, 128), jnp.float32),
            pltpu.SemaphoreType.DMA,
            pltpu.SemaphoreType.DMA,
            pltpu.SemaphoreType.DMA,
            pltpu.SemaphoreType.DMA,
        ],
        compiler_params=pltpu.CompilerParams(
            use_tc_tiling_on_sc=True, needs_layout_passes=False,
            disable_bounds_checks=True),
    )(_fmt_body)
    lines = fmt(emb_table_t, tail_lines)
    table_rm = lines.reshape(VOCAB_PAD, EMB)

    f = functools.partial(
        pl.kernel,
        out_type=jax.ShapeDtypeStruct((NW, B_PER_W, EMB), jnp.float32),
        mesh=_mesh(),
        scratch_types=[
            pltpu.VMEM((N_CHUNKS, IDX_PER_CHUNK), jnp.int32),
            pltpu.VMEM((NBUF, IDX_PER_CHUNK, EMB), jnp.float32),
            pltpu.VMEM((B_PER_W, EMB), jnp.float32),
            pltpu.SemaphoreType.DMA,
            pltpu.SemaphoreType.DMA,
            pltpu.SemaphoreType.DMA,
            pltpu.SemaphoreType.DMA,
        ],
        compiler_params=pltpu.CompilerParams(use_tc_tiling_on_sc=False),
    )(_cbow_body)
    return f(table_rm, x3)


def kernel(x, emb_table):
    x3 = x.astype(jnp.int32).reshape(NW, N_CHUNKS, IDX_PER_CHUNK)
    tail_lines = emb_table[N_FULL_BLOCKS * 128:].reshape(16, 128)
    out = _cbow(x3, emb_table.T, tail_lines)
    return out.reshape(BATCH, EMB)


# speedup vs baseline: 1.6568x; 1.6568x over previous
import functools

import jax
import jax.numpy as jnp
from jax import lax
from jax.experimental import pallas as pl
from jax.experimental.pallas import tpu as pltpu
from jax.experimental.pallas import tpu_sc as plsc

NC = 2
NS = 16
NW = NC * NS
LANES = 16

BATCH = 16384
CTX = 20
EMB = 32
VOCAB = 1_000_000
VOCAB_PAD = 1_000_064
N_BLOCKS = VOCAB_PAD // 128
N_FULL_BLOCKS = VOCAB // 128
BLK_PER_W = (N_BLOCKS + NW - 1) // NW
N_LINES = VOCAB_PAD * EMB // 128

B_PER_W = BATCH // NW
ROWS_PER_CHUNK = 4
IDX_PER_CHUNK = ROWS_PER_CHUNK * CTX
N_CHUNKS = B_PER_W // ROWS_PER_CHUNK
NBUF = 4
N_STEPS = N_CHUNKS // NBUF


def _transpose_block(in_v, out_v, iota16):
    e_vecs = (iota16, iota16 + LANES)

    def vblock(i, carry):
        v0 = i * LANES
        for d in range(LANES):
            v_vec = v0 + ((iota16 + d) & (LANES - 1))
            row_vec = v_vec >> 2
            col_base = (v_vec & 3) << 5
            for e_vec in e_vecs:
                vals = plsc.load_gather(in_v, [e_vec, v_vec])
                plsc.store_scatter(out_v, [row_vec, col_base + e_vec], vals)
        return carry

    lax.fori_loop(0, 128 // LANES, vblock, 0)


def _fmt_body(tab_t_hbm, tail_hbm, lines_hbm, in_v, out_v,
              isem0, isem1, osem0, osem1):
    wid = lax.axis_index("s") * NC + lax.axis_index("c")
    iota16 = lax.iota(jnp.int32, LANES)
    isems = (isem0, isem1)
    osems = (osem0, osem1)

    def blk(k):
        return jnp.minimum(k * NW + wid, N_FULL_BLOCKS - 1)

    def issue_in(k, b):
        c = blk(k)
        pltpu.async_copy(tab_t_hbm.at[:, pl.ds(c * 128, 128)],
                         in_v.at[b], isems[b])

    for b in range(2):
        issue_in(b, b)

    def step(g, carry):
        @pl.when(g >= 1)
        def _():
            for b in range(2):
                pltpu.make_async_copy(
                    out_v.at[b], lines_hbm.at[pl.ds(0, EMB)], osems[b]).wait()

        for b in range(2):
            k = g * 2 + b
            c = blk(k)
            pltpu.make_async_copy(
                tab_t_hbm.at[:, pl.ds(c * 128, 128)],
                in_v.at[b], isems[b]).wait()
            _transpose_block(in_v.at[b], out_v.at[b], iota16)
            pltpu.async_copy(
                out_v.at[b], lines_hbm.at[pl.ds(c * EMB, EMB)], osems[b])
            issue_in(k + 2, b)
        return carry

    lax.fori_loop(0, (BLK_PER_W + 1) // 2, step, 0)

    for b in range(2):
        pltpu.make_async_copy(
            out_v.at[b], lines_hbm.at[pl.ds(0, EMB)], osems[b]).wait()
        pltpu.make_async_copy(
            tab_t_hbm.at[:, pl.ds(0, 128)], in_v.at[b], isems[b]).wait()

    @pl.when(wid == 4)
    def _():
        pltpu.sync_copy(tail_hbm, in_v.at[0, pl.ds(0, 16)])
        pltpu.sync_copy(in_v.at[0, pl.ds(0, 16)],
                        lines_hbm.at[pl.ds(N_FULL_BLOCKS * EMB, 16)])


def _cbow_body(table_hbm, x_hbm, out_hbm, idx_v, rows_v, out_v,
               sem0, sem1, sem2, sem3):
    wid = lax.axis_index("s") * NC + lax.axis_index("c")
    sems = (sem0, sem1, sem2, sem3)

    pltpu.sync_copy(x_hbm.at[wid], idx_v)

    inv_ctx = jnp.float32(1.0 / CTX)

    for b in range(NBUF):
        pltpu.async_copy(table_hbm.at[idx_v.at[b]], rows_v.at[b], sems[b])

    def step(g, carry):
        for b in range(NBUF):
            j = g * NBUF + b
            pltpu.make_async_copy(
                table_hbm.at[idx_v.at[j]], rows_v.at[b], sems[b]).wait()
            for r in range(ROWS_PER_CHUNK):
                base = r * CTX
                lo = rows_v[b, base, pl.ds(0, LANES)]
                hi = rows_v[b, base, pl.ds(LANES, LANES)]
                for c in range(1, CTX):
                    lo = lo + rows_v[b, base + c, pl.ds(0, LANES)]
                    hi = hi + rows_v[b, base + c, pl.ds(LANES, LANES)]
                row = j * ROWS_PER_CHUNK + r
                out_v[row, pl.ds(0, LANES)] = lo * inv_ctx
                out_v[row, pl.ds(LANES, LANES)] = hi * inv_ctx

            @pl.when(g < N_STEPS - 1)
            def _():
                pltpu.async_copy(
                    table_hbm.at[idx_v.at[j + NBUF]], rows_v.at[b], sems[b])
        return carry

    lax.fori_loop(0, N_STEPS, step, 0)

    pltpu.sync_copy(out_v, out_hbm.at[wid])


def _mesh():
    return plsc.VectorSubcoreMesh(
        core_axis_name="c", subcore_axis_name="s",
        num_cores=NC, num_subcores=NS)


@jax.jit
def _cbow(x3, emb_table_t, tail_lines):
    fmt = functools.partial(
        pl.kernel,
        out_type=jax.ShapeDtypeStruct((N_LINES, 128), jnp.float32),
        mesh=_mesh(),
        scratch_types=[
            pltpu.VMEM((2, EMB, 128), jnp.float32),
            pltpu.VMEM((2, EMB, 128), jnp.float32),
            pltpu.SemaphoreType.DMA,
            pltpu.SemaphoreType.DMA,
            pltpu.SemaphoreType.DMA,
            pltpu.SemaphoreType.DMA,
        ],
        compiler_params=pltpu.CompilerParams(
            use_tc_tiling_on_sc=True, needs_layout_passes=False,
            disable_bounds_checks=True),
    )(_fmt_body)
    lines = fmt(emb_table_t, tail_lines)
    table_rm = lines.reshape(VOCAB_PAD, EMB)

    f = functools.partial(
        pl.kernel,
        out_type=jax.ShapeDtypeStruct((NW, B_PER_W, EMB), jnp.float32),
        mesh=_mesh(),
        scratch_types=[
            pltpu.VMEM((N_CHUNKS, IDX_PER_CHUNK), jnp.int32),
            pltpu.VMEM((NBUF, IDX_PER_CHUNK, EMB), jnp.float32),
            pltpu.VMEM((B_PER_W, EMB), jnp.float32),
            pltpu.SemaphoreType.DMA,
            pltpu.SemaphoreType.DMA,
            pltpu.SemaphoreType.DMA,
            pltpu.SemaphoreType.DMA,
        ],
        compiler_params=pltpu.CompilerParams(use_tc_tiling_on_sc=False),
    )(_cbow_body)
    return f(table_rm, x3)


def kernel(x, emb_table):
    x3 = x.astype(jnp.int32).reshape(NW, N_CHUNKS, IDX_PER_CHUNK)
    tail_lines = emb_table[N_FULL_BLOCKS * 128:].reshape(16, 128)
    out = _cbow(x3, emb_table.T, tail_lines)
    return out.reshape(BATCH, EMB)


# speedup vs baseline: 1.8142x; 1.0950x over previous
import functools

import jax
import jax.numpy as jnp
from jax import lax
from jax.experimental import pallas as pl
from jax.experimental.pallas import tpu as pltpu
from jax.experimental.pallas import tpu_sc as plsc

NC = 2
NS = 16
NW = NC * NS
LANES = 16

BATCH = 16384
CTX = 20
EMB = 32
VOCAB = 1_000_000
VOCAB_PAD = 1_000_064
N_BLOCKS = VOCAB_PAD // 128
N_FULL_BLOCKS = VOCAB // 128
BLK_PER_W = (N_BLOCKS + NW - 1) // NW
N_LINES = VOCAB_PAD * EMB // 128

B_PER_W = BATCH // NW
ROWS_PER_CHUNK = 4
IDX_PER_CHUNK = ROWS_PER_CHUNK * CTX
N_CHUNKS = B_PER_W // ROWS_PER_CHUNK
NBUF = 4
N_STEPS = N_CHUNKS // NBUF


def _transpose_block(in_v, out_v, iota16):
    e_vecs = (iota16, iota16 + LANES)

    def vblock(i, carry):
        v0 = i * LANES
        for d in range(LANES):
            v_vec = v0 + ((iota16 + d) & (LANES - 1))
            row_vec = v_vec >> 2
            col_base = (v_vec & 3) << 5
            for e_vec in e_vecs:
                vals = plsc.load_gather(in_v, [e_vec, v_vec])
                plsc.store_scatter(out_v, [row_vec, col_base + e_vec], vals)
        return carry

    lax.fori_loop(0, 128 // LANES, vblock, 0)


def _fmt_body(tab_t_hbm, tail_hbm, lines_hbm, in_v, out_v,
              isem0, isem1, osem0, osem1):
    wid = lax.axis_index("s") * NC + lax.axis_index("c")
    iota16 = lax.iota(jnp.int32, LANES)
    isems = (isem0, isem1)
    osems = (osem0, osem1)

    n = 244 + (wid < 4).astype(jnp.int32)

    def issue_in(k, b):
        c = k * NW + wid
        pltpu.async_copy(tab_t_hbm.at[:, pl.ds(c * 128, 128)],
                         in_v.at[b], isems[b])

    for b in range(2):
        issue_in(b, b)

    def step(g, carry):
        for b in range(2):
            k = g * 2 + b

            @pl.when(k < n)
            def _():
                c = k * NW + wid
                pltpu.make_async_copy(
                    tab_t_hbm.at[:, pl.ds(c * 128, 128)],
                    in_v.at[b], isems[b]).wait()

                @pl.when(k >= 2)
                def _():
                    pltpu.make_async_copy(
                        out_v.at[b], lines_hbm.at[pl.ds(c * EMB, EMB)],
                        osems[b]).wait()

                _transpose_block(in_v.at[b], out_v.at[b], iota16)
                pltpu.async_copy(
                    out_v.at[b], lines_hbm.at[pl.ds(c * EMB, EMB)], osems[b])

                @pl.when(k + 2 < n)
                def _():
                    issue_in(k + 2, b)
        return carry

    lax.fori_loop(0, (BLK_PER_W + 1) // 2, step, 0)

    for b in range(2):
        pltpu.make_async_copy(
            out_v.at[b], lines_hbm.at[pl.ds(0, EMB)], osems[b]).wait()

    @pl.when(wid == 4)
    def _():
        pltpu.sync_copy(tail_hbm, in_v.at[0, pl.ds(0, 16)])
        pltpu.sync_copy(in_v.at[0, pl.ds(0, 16)],
                        lines_hbm.at[pl.ds(N_FULL_BLOCKS * EMB, 16)])


def _cbow_body(table_hbm, x_hbm, out_hbm, idx_v, rows_v, out_v,
               sem0, sem1, sem2, sem3):
    wid = lax.axis_index("s") * NC + lax.axis_index("c")
    sems = (sem0, sem1, sem2, sem3)

    pltpu.sync_copy(x_hbm.at[wid], idx_v)

    inv_ctx = jnp.float32(1.0 / CTX)

    for b in range(NBUF):
        pltpu.async_copy(table_hbm.at[idx_v.at[b]], rows_v.at[b], sems[b])

    def step(g, carry):
        for b in range(NBUF):
            j = g * NBUF + b
            pltpu.make_async_copy(
                table_hbm.at[idx_v.at[j]], rows_v.at[b], sems[b]).wait()
            for r in range(ROWS_PER_CHUNK):
                base = r * CTX
                lo = rows_v[b, base, pl.ds(0, LANES)]
                hi = rows_v[b, base, pl.ds(LANES, LANES)]
                for c in range(1, CTX):
                    lo = lo + rows_v[b, base + c, pl.ds(0, LANES)]
                    hi = hi + rows_v[b, base + c, pl.ds(LANES, LANES)]
                row = j * ROWS_PER_CHUNK + r
                out_v[row, pl.ds(0, LANES)] = lo * inv_ctx
                out_v[row, pl.ds(LANES, LANES)] = hi * inv_ctx

            @pl.when(g < N_STEPS - 1)
            def _():
                pltpu.async_copy(
                    table_hbm.at[idx_v.at[j + NBUF]], rows_v.at[b], sems[b])
        return carry

    lax.fori_loop(0, N_STEPS, step, 0)

    pltpu.sync_copy(out_v, out_hbm.at[wid])


def _mesh():
    return plsc.VectorSubcoreMesh(
        core_axis_name="c", subcore_axis_name="s",
        num_cores=NC, num_subcores=NS)


@jax.jit
def _cbow(x3, emb_table_t, tail_lines):
    fmt = functools.partial(
        pl.kernel,
        out_type=jax.ShapeDtypeStruct((N_LINES, 128), jnp.float32),
        mesh=_mesh(),
        scratch_types=[
            pltpu.VMEM((2, EMB, 128), jnp.float32),
            pltpu.VMEM((2, EMB, 128), jnp.float32),
            pltpu.SemaphoreType.DMA,
            pltpu.SemaphoreType.DMA,
            pltpu.SemaphoreType.DMA,
            pltpu.SemaphoreType.DMA,
        ],
        compiler_params=pltpu.CompilerParams(
            use_tc_tiling_on_sc=True, needs_layout_passes=False),
    )(_fmt_body)
    lines = fmt(emb_table_t, tail_lines)
    table_rm = lines.reshape(VOCAB_PAD, EMB)

    f = functools.partial(
        pl.kernel,
        out_type=jax.ShapeDtypeStruct((NW, B_PER_W, EMB), jnp.float32),
        mesh=_mesh(),
        scratch_types=[
            pltpu.VMEM((N_CHUNKS, IDX_PER_CHUNK), jnp.int32),
            pltpu.VMEM((NBUF, IDX_PER_CHUNK, EMB), jnp.float32),
            pltpu.VMEM((B_PER_W, EMB), jnp.float32),
            pltpu.SemaphoreType.DMA,
            pltpu.SemaphoreType.DMA,
            pltpu.SemaphoreType.DMA,
            pltpu.SemaphoreType.DMA,
        ],
        compiler_params=pltpu.CompilerParams(use_tc_tiling_on_sc=False),
    )(_cbow_body)
    return f(table_rm, x3)


def kernel(x, emb_table):
    x3 = x.astype(jnp.int32).reshape(NW, N_CHUNKS, IDX_PER_CHUNK)
    tail_lines = emb_table[N_FULL_BLOCKS * 128:].reshape(16, 128)
    out = _cbow(x3, emb_table.T, tail_lines)
    return out.reshape(BATCH, EMB)
